# baseline (speedup 1.0000x reference)
import jax
import jax.numpy as jnp
from jax.experimental import pallas as pl

B = 16384
D = 128
GRID = 20
BLK = 2048


def _fused(x_ref, we0, be0, we1, be1, we2, be2, we3, be3,
           wd0, bd0, wd1, bd1, wd2, bd2, wd3, bd3,
           ew0, eb0, ew1, eb1, somw,
           code_out, xp_out, cosim_out, z_out, gamma_out):
    eps = 1e-8
    xT = x_ref[...].T
    h = jnp.tanh(we0[...] @ xT + be0[...])
    h = jnp.tanh(we1[...] @ h + be1[...])
    h = jnp.tanh(we2[...] @ h + be2[...])
    codeT = we3[...] @ h + be3[...]
    g = jnp.tanh(wd0[...] @ codeT + bd0[...])
    g = jnp.tanh(wd1[...] @ g + bd1[...])
    g = jnp.tanh(wd2[...] @ g + bd2[...])
    xpT = wd3[...] @ g + bd3[...]

    diff = xT - xpT
    nx2 = jnp.sum(xT * xT, axis=0, keepdims=True)
    dot = jnp.sum(xT * xpT, axis=0, keepdims=True)
    nxp2 = jnp.sum(xpT * xpT, axis=0, keepdims=True)
    e2 = jnp.sum(diff * diff, axis=0, keepdims=True)
    nx = jnp.sqrt(nx2)
    cosim = dot / (nx * jnp.sqrt(nxp2) + eps)
    euclid = jnp.sqrt(e2) / (nx + eps)

    sw = somw[...]
    swsq = jnp.sum(sw * sw, axis=1)[:, None]
    d2 = nx2 - 2.0 * (sw @ xT) + swsq
    idx = jnp.argmin(d2, axis=0).reshape(1, BLK)
    zi = (idx // GRID).astype(jnp.float32)
    zj = (idx % GRID).astype(jnp.float32)

    zT = jnp.concatenate([codeT, cosim, euclid,
                          zi / 20.0, zj / 20.0], axis=0)

    e = jnp.tanh(ew0[...] @ zT + eb0[...])
    logits = ew1[...] @ e + eb1[...]
    m = jnp.max(logits, axis=0, keepdims=True)
    ex = jnp.exp(logits - m)
    gammaT = ex / jnp.sum(ex, axis=0, keepdims=True)

    tail = jnp.concatenate([zT, gammaT, cosim], axis=0).T
    xp_out[...] = xpT.T
    code_out[...] = tail[:, 0:2]
    z_out[...] = tail[:, 0:6]
    gamma_out[...] = tail[:, 6:10]
    cosim_out[...] = tail[:, 10]


def kernel(X, We0, be0, We1, be1, We2, be2, We3, be3,
           Wd0, bd0, Wd1, bd1, Wd2, bd2, Wd3, bd3,
           Ew0, Eb0, Ew1, Eb1, som_w):
    f32 = jnp.float32
    grid = B // BLK

    wTs = [w.T for w in (We0, We1, We2, We3, Wd0, Wd1, Wd2, Wd3, Ew0, Ew1)]
    bTs = [b.reshape(-1, 1) for b in (be0, be1, be2, be3,
                                      bd0, bd1, bd2, bd3, Eb0, Eb1)]
    (we0T, we1T, we2T, we3T, wd0T, wd1T, wd2T, wd3T, ew0T, ew1T) = wTs
    (be0c, be1c, be2c, be3c, bd0c, bd1c, bd2c, bd3c, eb0c, eb1c) = bTs

    def full(a):
        return pl.BlockSpec(a.shape, lambda i: (0,) * a.ndim)

    in_arrays = (X, we0T, be0c, we1T, be1c, we2T, be2c, we3T, be3c,
                 wd0T, bd0c, wd1T, bd1c, wd2T, bd2c, wd3T, bd3c,
                 ew0T, eb0c, ew1T, eb1c, som_w)
    in_specs = [pl.BlockSpec((BLK, D), lambda i: (i, 0))]
    in_specs += [full(a) for a in in_arrays[1:]]

    out_shape = (
        jax.ShapeDtypeStruct((B, 2), f32),
        jax.ShapeDtypeStruct((B, D), f32),
        jax.ShapeDtypeStruct((B,), f32),
        jax.ShapeDtypeStruct((B, 6), f32),
        jax.ShapeDtypeStruct((B, 4), f32),
    )
    out_specs = (
        pl.BlockSpec((BLK, 2), lambda i: (i, 0)),
        pl.BlockSpec((BLK, D), lambda i: (i, 0)),
        pl.BlockSpec((BLK,), lambda i: (i,)),
        pl.BlockSpec((BLK, 6), lambda i: (i, 0)),
        pl.BlockSpec((BLK, 4), lambda i: (i, 0)),
    )

    return pl.pallas_call(
        _fused,
        grid=(grid,),
        in_specs=in_specs,
        out_specs=out_specs,
        out_shape=out_shape,
    )(*in_arrays)


# speedup vs baseline: 1.2162x; 1.2162x over previous
import jax
import jax.numpy as jnp
from jax import lax
from jax.experimental import pallas as pl

B = 16384
D = 128
GRID = 20
BLK = 2048

_DN = (((0,), (0,)), ((), ()))


def _wmm(w_ref, h):
    return lax.dot_general(w_ref[...], h, _DN)


def _fused(x_ref, we0, be0, we1, be1, we2, be2, we3, be3,
           wd0, bd0, wd1, bd1, wd2, bd2, wd3, bd3,
           ew0, eb0, ew1, eb1, somw,
           code_out, xp_out, cosim_out, z_out, gamma_out):
    eps = 1e-8
    xT = x_ref[...].T
    h = jnp.tanh(_wmm(we0, xT) + be0[...][:, None])
    h = jnp.tanh(_wmm(we1, h) + be1[...][:, None])
    h = jnp.tanh(_wmm(we2, h) + be2[...][:, None])
    codeT = _wmm(we3, h) + be3[...][:, None]
    g = jnp.tanh(_wmm(wd0, codeT) + bd0[...][:, None])
    g = jnp.tanh(_wmm(wd1, g) + bd1[...][:, None])
    g = jnp.tanh(_wmm(wd2, g) + bd2[...][:, None])
    xpT = _wmm(wd3, g) + bd3[...][:, None]

    diff = xT - xpT
    nx2 = jnp.sum(xT * xT, axis=0, keepdims=True)
    dot = jnp.sum(xT * xpT, axis=0, keepdims=True)
    nxp2 = jnp.sum(xpT * xpT, axis=0, keepdims=True)
    e2 = jnp.sum(diff * diff, axis=0, keepdims=True)
    nx = jnp.sqrt(nx2)
    cosim = dot / (nx * jnp.sqrt(nxp2) + eps)
    euclid = jnp.sqrt(e2) / (nx + eps)

    sw = somw[...]
    swsq = jnp.sum(sw * sw, axis=1)[:, None]
    d2 = nx2 - 2.0 * (sw @ xT) + swsq
    idx = jnp.argmin(d2, axis=0).reshape(1, BLK)
    zi = (idx // GRID).astype(jnp.float32)
    zj = (idx % GRID).astype(jnp.float32)

    zT = jnp.concatenate([codeT, cosim, euclid,
                          zi / 20.0, zj / 20.0], axis=0)

    e = jnp.tanh(_wmm(ew0, zT) + eb0[...][:, None])
    logits = _wmm(ew1, e) + eb1[...][:, None]
    m = jnp.max(logits, axis=0, keepdims=True)
    ex = jnp.exp(logits - m)
    gammaT = ex / jnp.sum(ex, axis=0, keepdims=True)

    tail = jnp.concatenate([zT, gammaT, cosim], axis=0).T
    xp_out[...] = xpT.T
    code_out[...] = tail[:, 0:2]
    z_out[...] = tail[:, 0:6]
    gamma_out[...] = tail[:, 6:10]
    cosim_out[...] = tail[:, 10]


def kernel(X, We0, be0, We1, be1, We2, be2, We3, be3,
           Wd0, bd0, Wd1, bd1, Wd2, bd2, Wd3, bd3,
           Ew0, Eb0, Ew1, Eb1, som_w):
    f32 = jnp.float32
    grid = B // BLK

    def full(a):
        return pl.BlockSpec(a.shape, lambda i: (0,) * a.ndim)

    in_arrays = (X, We0, be0, We1, be1, We2, be2, We3, be3,
                 Wd0, bd0, Wd1, bd1, Wd2, bd2, Wd3, bd3,
                 Ew0, Eb0, Ew1, Eb1, som_w)
    in_specs = [pl.BlockSpec((BLK, D), lambda i: (i, 0))]
    in_specs += [full(a) for a in in_arrays[1:]]

    out_shape = (
        jax.ShapeDtypeStruct((B, 2), f32),
        jax.ShapeDtypeStruct((B, D), f32),
        jax.ShapeDtypeStruct((B,), f32),
        jax.ShapeDtypeStruct((B, 6), f32),
        jax.ShapeDtypeStruct((B, 4), f32),
    )
    out_specs = (
        pl.BlockSpec((BLK, 2), lambda i: (i, 0)),
        pl.BlockSpec((BLK, D), lambda i: (i, 0)),
        pl.BlockSpec((BLK,), lambda i: (i,)),
        pl.BlockSpec((BLK, 6), lambda i: (i, 0)),
        pl.BlockSpec((BLK, 4), lambda i: (i, 0)),
    )

    return pl.pallas_call(
        _fused,
        grid=(grid,),
        in_specs=in_specs,
        out_specs=out_specs,
        out_shape=out_shape,
    )(*in_arrays)


# speedup vs baseline: 2.2712x; 1.8675x over previous
import jax
import jax.numpy as jnp
from jax import lax
from jax.experimental import pallas as pl

B = 16384
D = 128
GRID = 20
BLK = 2048

_DN = (((0,), (0,)), ((), ()))


def _wmm(w_ref, h):
    return lax.dot_general(w_ref[...], h, _DN)


def _fused(x_ref, we0, be0, we1, be1, we2, be2, we3, be3,
           wd0, bd0, wd1, bd1, wd2, bd2, wd3, bd3,
           ew0, eb0, ew1, eb1, somw,
           code_out, xp_out, cosim_out, z_out, gamma_out):
    eps = 1e-8
    xT = x_ref[...].T
    h = jnp.tanh(_wmm(we0, xT) + be0[...][:, None])
    h = jnp.tanh(_wmm(we1, h) + be1[...][:, None])
    h = jnp.tanh(_wmm(we2, h) + be2[...][:, None])
    codeT = _wmm(we3, h) + be3[...][:, None]
    g = jnp.tanh(_wmm(wd0, codeT) + bd0[...][:, None])
    g = jnp.tanh(_wmm(wd1, g) + bd1[...][:, None])
    g = jnp.tanh(_wmm(wd2, g) + bd2[...][:, None])
    xpT = _wmm(wd3, g) + bd3[...][:, None]

    diff = xT - xpT
    nx2 = jnp.sum(xT * xT, axis=0, keepdims=True)
    dot = jnp.sum(xT * xpT, axis=0, keepdims=True)
    nxp2 = jnp.sum(xpT * xpT, axis=0, keepdims=True)
    e2 = jnp.sum(diff * diff, axis=0, keepdims=True)
    nx = jnp.sqrt(nx2)
    cosim = dot / (nx * jnp.sqrt(nxp2) + eps)
    euclid = jnp.sqrt(e2) / (nx + eps)

    sw = somw[...]
    swsq = jnp.sum(sw * sw, axis=1)[:, None]
    d2 = nx2 - 2.0 * (sw @ xT) + swsq
    idx = jnp.argmin(d2, axis=0).reshape(1, BLK)
    zi = (idx // GRID).astype(jnp.float32)
    zj = (idx % GRID).astype(jnp.float32)

    zT = jnp.concatenate([codeT, cosim, euclid,
                          zi / 20.0, zj / 20.0], axis=0)

    e = jnp.tanh(_wmm(ew0, zT) + eb0[...][:, None])
    logits = _wmm(ew1, e) + eb1[...][:, None]
    m = jnp.max(logits, axis=0, keepdims=True)
    ex = jnp.exp(logits - m)
    gammaT = ex / jnp.sum(ex, axis=0, keepdims=True)

    xp_out[...] = xpT.T
    code_out[...] = codeT
    z_out[...] = zT
    gamma_out[...] = gammaT
    cosim_out[...] = cosim


def kernel(X, We0, be0, We1, be1, We2, be2, We3, be3,
           Wd0, bd0, Wd1, bd1, Wd2, bd2, Wd3, bd3,
           Ew0, Eb0, Ew1, Eb1, som_w):
    f32 = jnp.float32
    grid = B // BLK

    def full(a):
        return pl.BlockSpec(a.shape, lambda i: (0,) * a.ndim)

    in_arrays = (X, We0, be0, We1, be1, We2, be2, We3, be3,
                 Wd0, bd0, Wd1, bd1, Wd2, bd2, Wd3, bd3,
                 Ew0, Eb0, Ew1, Eb1, som_w)
    in_specs = [pl.BlockSpec((BLK, D), lambda i: (i, 0))]
    in_specs += [full(a) for a in in_arrays[1:]]

    out_shape = (
        jax.ShapeDtypeStruct((2, B), f32),
        jax.ShapeDtypeStruct((B, D), f32),
        jax.ShapeDtypeStruct((1, B), f32),
        jax.ShapeDtypeStruct((6, B), f32),
        jax.ShapeDtypeStruct((4, B), f32),
    )
    out_specs = (
        pl.BlockSpec((2, BLK), lambda i: (0, i)),
        pl.BlockSpec((BLK, D), lambda i: (i, 0)),
        pl.BlockSpec((1, BLK), lambda i: (0, i)),
        pl.BlockSpec((6, BLK), lambda i: (0, i)),
        pl.BlockSpec((4, BLK), lambda i: (0, i)),
    )

    codeT, x_prime, cosim_row, zT, gammaT = pl.pallas_call(
        _fused,
        grid=(grid,),
        in_specs=in_specs,
        out_specs=out_specs,
        out_shape=out_shape,
    )(*in_arrays)
    return (codeT.T, x_prime, cosim_row.reshape(B), zT.T, gammaT.T)


# speedup vs baseline: 2.6032x; 1.1462x over previous
import jax
import jax.numpy as jnp
from jax import lax
from jax.experimental import pallas as pl

B = 16384
D = 128
GRID = 20
BLK = 4096

_DN = (((0,), (0,)), ((), ()))


def _wmm(w_ref, h):
    return lax.dot_general(w_ref[...], h, _DN)


def _fused(x_ref, we0, be0, we1, be1, we2, be2, we3, be3,
           wd0, bd0, wd1, bd1, wd2, bd2, wd3, bd3,
           ew0, eb0, ew1, eb1, somw,
           code_out, xp_out, cosim_out, z_out, gamma_out):
    eps = 1e-8
    xT = x_ref[...].T
    h = jnp.tanh(_wmm(we0, xT) + be0[...][:, None])
    h = jnp.tanh(_wmm(we1, h) + be1[...][:, None])
    h = jnp.tanh(_wmm(we2, h) + be2[...][:, None])
    codeT = _wmm(we3, h) + be3[...][:, None]
    g = jnp.tanh(_wmm(wd0, codeT) + bd0[...][:, None])
    g = jnp.tanh(_wmm(wd1, g) + bd1[...][:, None])
    g = jnp.tanh(_wmm(wd2, g) + bd2[...][:, None])
    xpT = _wmm(wd3, g) + bd3[...][:, None]

    diff = xT - xpT
    nx2 = jnp.sum(xT * xT, axis=0, keepdims=True)
    dot = jnp.sum(xT * xpT, axis=0, keepdims=True)
    nxp2 = jnp.sum(xpT * xpT, axis=0, keepdims=True)
    e2 = jnp.sum(diff * diff, axis=0, keepdims=True)
    nx = jnp.sqrt(nx2)
    cosim = dot / (nx * jnp.sqrt(nxp2) + eps)
    euclid = jnp.sqrt(e2) / (nx + eps)

    sw = somw[...]
    swsq = jnp.sum(sw * sw, axis=1)[:, None]
    d2 = nx2 - 2.0 * (sw @ xT) + swsq
    idx = jnp.argmin(d2, axis=0).reshape(1, BLK)
    zi = (idx // GRID).astype(jnp.float32)
    zj = (idx % GRID).astype(jnp.float32)

    zT = jnp.concatenate([codeT, cosim, euclid,
                          zi / 20.0, zj / 20.0], axis=0)

    e = jnp.tanh(_wmm(ew0, zT) + eb0[...][:, None])
    logits = _wmm(ew1, e) + eb1[...][:, None]
    m = jnp.max(logits, axis=0, keepdims=True)
    ex = jnp.exp(logits - m)
    gammaT = ex / jnp.sum(ex, axis=0, keepdims=True)

    xp_out[...] = xpT.T
    code_out[...] = codeT
    z_out[...] = zT
    gamma_out[...] = gammaT
    cosim_out[...] = cosim


def kernel(X, We0, be0, We1, be1, We2, be2, We3, be3,
           Wd0, bd0, Wd1, bd1, Wd2, bd2, Wd3, bd3,
           Ew0, Eb0, Ew1, Eb1, som_w):
    f32 = jnp.float32
    grid = B // BLK

    def full(a):
        return pl.BlockSpec(a.shape, lambda i: (0,) * a.ndim)

    in_arrays = (X, We0, be0, We1, be1, We2, be2, We3, be3,
                 Wd0, bd0, Wd1, bd1, Wd2, bd2, Wd3, bd3,
                 Ew0, Eb0, Ew1, Eb1, som_w)
    in_specs = [pl.BlockSpec((BLK, D), lambda i: (i, 0))]
    in_specs += [full(a) for a in in_arrays[1:]]

    out_shape = (
        jax.ShapeDtypeStruct((2, B), f32),
        jax.ShapeDtypeStruct((B, D), f32),
        jax.ShapeDtypeStruct((1, B), f32),
        jax.ShapeDtypeStruct((6, B), f32),
        jax.ShapeDtypeStruct((4, B), f32),
    )
    out_specs = (
        pl.BlockSpec((2, BLK), lambda i: (0, i)),
        pl.BlockSpec((BLK, D), lambda i: (i, 0)),
        pl.BlockSpec((1, BLK), lambda i: (0, i)),
        pl.BlockSpec((6, BLK), lambda i: (0, i)),
        pl.BlockSpec((4, BLK), lambda i: (0, i)),
    )

    codeT, x_prime, cosim_row, zT, gammaT = pl.pallas_call(
        _fused,
        grid=(grid,),
        in_specs=in_specs,
        out_specs=out_specs,
        out_shape=out_shape,
    )(*in_arrays)
    return (codeT.T, x_prime, cosim_row.reshape(B), zT.T, gammaT.T)


# speedup vs baseline: 2.6678x; 1.0248x over previous
import jax
import jax.numpy as jnp
from jax import lax
from jax.experimental import pallas as pl

B = 16384
D = 128
GRID = 20
BLK = 8192

_DN = (((0,), (0,)), ((), ()))


def _wmm(w_ref, h):
    return lax.dot_general(w_ref[...], h, _DN)


def _fused(x_ref, we0, be0, we1, be1, we2, be2, we3, be3,
           wd0, bd0, wd1, bd1, wd2, bd2, wd3, bd3,
           ew0, eb0, ew1, eb1, somw,
           code_out, xp_out, cosim_out, z_out, gamma_out):
    eps = 1e-8
    xT = x_ref[...].T
    h = jnp.tanh(_wmm(we0, xT) + be0[...][:, None])
    h = jnp.tanh(_wmm(we1, h) + be1[...][:, None])
    h = jnp.tanh(_wmm(we2, h) + be2[...][:, None])
    codeT = _wmm(we3, h) + be3[...][:, None]
    g = jnp.tanh(_wmm(wd0, codeT) + bd0[...][:, None])
    g = jnp.tanh(_wmm(wd1, g) + bd1[...][:, None])
    g = jnp.tanh(_wmm(wd2, g) + bd2[...][:, None])
    xpT = _wmm(wd3, g) + bd3[...][:, None]

    diff = xT - xpT
    nx2 = jnp.sum(xT * xT, axis=0, keepdims=True)
    dot = jnp.sum(xT * xpT, axis=0, keepdims=True)
    nxp2 = jnp.sum(xpT * xpT, axis=0, keepdims=True)
    e2 = jnp.sum(diff * diff, axis=0, keepdims=True)
    nx = jnp.sqrt(nx2)
    cosim = dot / (nx * jnp.sqrt(nxp2) + eps)
    euclid = jnp.sqrt(e2) / (nx + eps)

    sw = somw[...]
    swsq = jnp.sum(sw * sw, axis=1)[:, None]
    d2 = nx2 - 2.0 * (sw @ xT) + swsq
    idx = jnp.argmin(d2, axis=0).reshape(1, BLK)
    zi = (idx // GRID).astype(jnp.float32)
    zj = (idx % GRID).astype(jnp.float32)

    zT = jnp.concatenate([codeT, cosim, euclid,
                          zi / 20.0, zj / 20.0], axis=0)

    e = jnp.tanh(_wmm(ew0, zT) + eb0[...][:, None])
    logits = _wmm(ew1, e) + eb1[...][:, None]
    m = jnp.max(logits, axis=0, keepdims=True)
    ex = jnp.exp(logits - m)
    gammaT = ex / jnp.sum(ex, axis=0, keepdims=True)

    xp_out[...] = xpT.T
    code_out[...] = codeT
    z_out[...] = zT
    gamma_out[...] = gammaT
    cosim_out[...] = cosim


def kernel(X, We0, be0, We1, be1, We2, be2, We3, be3,
           Wd0, bd0, Wd1, bd1, Wd2, bd2, Wd3, bd3,
           Ew0, Eb0, Ew1, Eb1, som_w):
    f32 = jnp.float32
    grid = B // BLK

    def full(a):
        return pl.BlockSpec(a.shape, lambda i: (0,) * a.ndim)

    in_arrays = (X, We0, be0, We1, be1, We2, be2, We3, be3,
                 Wd0, bd0, Wd1, bd1, Wd2, bd2, Wd3, bd3,
                 Ew0, Eb0, Ew1, Eb1, som_w)
    in_specs = [pl.BlockSpec((BLK, D), lambda i: (i, 0))]
    in_specs += [full(a) for a in in_arrays[1:]]

    out_shape = (
        jax.ShapeDtypeStruct((2, B), f32),
        jax.ShapeDtypeStruct((B, D), f32),
        jax.ShapeDtypeStruct((1, B), f32),
        jax.ShapeDtypeStruct((6, B), f32),
        jax.ShapeDtypeStruct((4, B), f32),
    )
    out_specs = (
        pl.BlockSpec((2, BLK), lambda i: (0, i)),
        pl.BlockSpec((BLK, D), lambda i: (i, 0)),
        pl.BlockSpec((1, BLK), lambda i: (0, i)),
        pl.BlockSpec((6, BLK), lambda i: (0, i)),
        pl.BlockSpec((4, BLK), lambda i: (0, i)),
    )

    codeT, x_prime, cosim_row, zT, gammaT = pl.pallas_call(
        _fused,
        grid=(grid,),
        in_specs=in_specs,
        out_specs=out_specs,
        out_shape=out_shape,
    )(*in_arrays)
    return (codeT.T, x_prime, cosim_row.reshape(B), zT.T, gammaT.T)
